# Initial kernel scaffold; baseline (speedup 1.0000x reference)
import jax
import jax.numpy as jnp
from jax.experimental import pallas as pl


def kernel(h, edge_index, edge_attr, We, be, eps, W1, g1, b1, W2, g2, b2):
    raise NotImplementedError("write your pallas kernel here")


# speedup vs baseline: 1.3475x; 1.3475x over previous
import jax
import jax.numpy as jnp
from jax import lax
from jax.experimental import pallas as pl
from jax.experimental.pallas import tpu as pltpu
from jax.experimental.pallas import tpu_sc as plsc

HIDDEN = 128
EDGE_DIM = 16
N_NODES = 10000
N_EDGES = 320000
BN_EPS = 1e-5

NS = 16
NC = 2
TILES = NC * NS
EPT = 10240
E_PAD = EPT * TILES
CHUNK = 128
KS = CHUNK // 128
NCHUNK = EPT // CHUNK
SROWS = 632
LROWS = N_NODES - (NS - 1) * SROWS

BE = 1280
NEB_REAL = N_EDGES // BE
NEB = E_PAD // BE


def _ea_body(x_ref, we_ref, be_ref, o_ref):
    i = pl.program_id(0)
    v = jnp.dot(x_ref[...], we_ref[...], preferred_element_type=jnp.float32)
    v = v + be_ref[...][None, :]
    rows = i * BE + lax.broadcasted_iota(jnp.int32, (BE, HIDDEN), 0)
    o_ref[...] = jnp.where(rows < N_EDGES, v, -1e30)


def _edge_linear(edge_attr, We, be):
    return pl.pallas_call(
        _ea_body,
        grid=(NEB,),
        in_specs=[
            pl.BlockSpec((BE, EDGE_DIM), lambda i: (jnp.minimum(i, NEB_REAL - 1), 0)),
            pl.BlockSpec((EDGE_DIM, HIDDEN), lambda i: (0, 0)),
            pl.BlockSpec((HIDDEN,), lambda i: (0,)),
        ],
        out_specs=pl.BlockSpec((BE, HIDDEN), lambda i: (i, 0)),
        out_shape=jax.ShapeDtypeStruct((E_PAD, HIDDEN), jnp.float32),
    )(edge_attr, We, be)


def _sc_body(h_hbm, init, srcr, dstr, ea, out, z_sh, src_v, dst_v, ea_v, gat_v, sem):
    c = lax.axis_index("c")
    s = lax.axis_index("s")
    wid = c * NS + s
    r0 = s * SROWS

    @pl.when(s < NS - 1)
    def _stage_full():
        pltpu.sync_copy(init.at[c, pl.ds(r0, SROWS)], z_sh.at[pl.ds(r0, SROWS)])

    @pl.when(s == NS - 1)
    def _stage_last():
        pltpu.sync_copy(init.at[c, pl.ds(r0, LROWS)], z_sh.at[pl.ds(r0, LROWS)])

    plsc.subcore_barrier()

    def chunk_body(g, carry):
        e0 = wid * EPT + g * CHUNK
        ci = wid * NCHUNK + g
        pltpu.sync_copy(srcr.at[ci], src_v)
        pltpu.sync_copy(dstr.at[ci], dst_v)
        pltpu.sync_copy(ea.at[pl.ds(e0, CHUNK)], ea_v)
        handles = []
        for j in range(KS):
            handles.append(pltpu.async_copy(
                h_hbm.at[src_v.at[j]], gat_v.at[pl.ds(j * 128, 128)], sem))
        for hd in handles:
            hd.wait()

        def vbody(i, acc):
            g_r = gat_v.at[i]
            e_r = ea_v.at[i]
            for k in range(HIDDEN // 16):
                sl = pl.ds(k * 16, 16)
                e_r[sl] = jnp.maximum(g_r[sl] + e_r[sl], 0.0)
            return acc
        lax.fori_loop(0, CHUNK, vbody, 0, unroll=2)

        for j in range(KS):
            pltpu.sync_copy(ea_v.at[pl.ds(j * 128, 128)],
                            z_sh.at[dst_v.at[j]], add=True)
        return carry

    lax.fori_loop(0, NCHUNK, chunk_body, 0)
    plsc.subcore_barrier()

    @pl.when(s < NS - 1)
    def _out_full():
        pltpu.sync_copy(z_sh.at[pl.ds(r0, SROWS)], out.at[c, pl.ds(r0, SROWS)])

    @pl.when(s == NS - 1)
    def _out_last():
        pltpu.sync_copy(z_sh.at[pl.ds(r0, LROWS)], out.at[c, pl.ds(r0, LROWS)])


def _sc_aggregate(h, init, srcr, dstr, ea):
    mesh = plsc.VectorSubcoreMesh(core_axis_name="c", subcore_axis_name="s")
    return pl.kernel(
        _sc_body,
        out_type=jax.ShapeDtypeStruct((NC, N_NODES, HIDDEN), jnp.float32),
        mesh=mesh,
        scratch_types=[
            pltpu.VMEM_SHARED((N_NODES, HIDDEN), jnp.float32),
            pltpu.VMEM((KS, 128), jnp.int32),
            pltpu.VMEM((KS, 128), jnp.int32),
            pltpu.VMEM((CHUNK, HIDDEN), jnp.float32),
            pltpu.VMEM((CHUNK, HIDDEN), jnp.float32),
            pltpu.SemaphoreType.DMA,
        ],
    )(h, init, srcr, dstr, ea)


def _mlp_body(pa_ref, pb_ref, h_ref, w1_ref, g1_ref, b1_ref, w2_ref,
              g2_ref, b2_ref, o_ref):
    z = pa_ref[0] + pb_ref[0]
    y = jnp.dot(z, w1_ref[...], preferred_element_type=jnp.float32)
    mu = jnp.mean(y, axis=0, keepdims=True)
    d = y - mu
    var = jnp.mean(d * d, axis=0, keepdims=True)
    y = d * lax.rsqrt(var + BN_EPS) * g1_ref[...][None, :] + b1_ref[...][None, :]
    y = jnp.maximum(y, 0.0)
    y = jnp.dot(y, w2_ref[...], preferred_element_type=jnp.float32)
    mu = jnp.mean(y, axis=0, keepdims=True)
    d = y - mu
    var = jnp.mean(d * d, axis=0, keepdims=True)
    y = d * lax.rsqrt(var + BN_EPS) * g2_ref[...][None, :] + b2_ref[...][None, :]
    o_ref[...] = jnp.maximum(y, 0.0) + h_ref[...]


def _mlp(z2, h, W1, g1, b1, W2, g2, b2):
    full = lambda *shape: pl.BlockSpec(shape, lambda: (0,) * len(shape))
    return pl.pallas_call(
        _mlp_body,
        in_specs=[
            full(1, N_NODES, HIDDEN), full(1, N_NODES, HIDDEN),
            full(N_NODES, HIDDEN),
            full(HIDDEN, 2 * HIDDEN), full(2 * HIDDEN), full(2 * HIDDEN),
            full(2 * HIDDEN, HIDDEN), full(HIDDEN), full(HIDDEN),
        ],
        out_specs=full(N_NODES, HIDDEN),
        out_shape=jax.ShapeDtypeStruct((N_NODES, HIDDEN), jnp.float32),
    )(z2[0:1], z2[1:2], h, W1, g1, b1, W2, g2, b2)


def kernel(h, edge_index, edge_attr, We, be, eps, W1, g1, b1, W2, g2, b2):
    src = edge_index[0].astype(jnp.int32)
    dst = edge_index[1].astype(jnp.int32)
    pad = E_PAD - N_EDGES
    zpad = jnp.zeros((pad,), jnp.int32)
    srcr = jnp.concatenate([src, zpad]).reshape(TILES * NCHUNK, KS, 128)
    dstr = jnp.concatenate([dst, zpad]).reshape(TILES * NCHUNK, KS, 128)

    init = jnp.stack([(1.0 + eps) * h, jnp.zeros_like(h)])

    ea = _edge_linear(edge_attr, We, be)
    z2 = _sc_aggregate(h, init, srcr, dstr, ea)
    return _mlp(z2, h, W1, g1, b1, W2, g2, b2)


# speedup vs baseline: 1.5044x; 1.1164x over previous
import jax
import jax.numpy as jnp
from jax import lax
from jax.experimental import pallas as pl
from jax.experimental.pallas import tpu as pltpu
from jax.experimental.pallas import tpu_sc as plsc

HIDDEN = 128
EDGE_DIM = 16
N_NODES = 10000
N_EDGES = 320000
BN_EPS = 1e-5

NS = 16
NC = 2
TILES = NC * NS
EPT = 10240
E_PAD = EPT * TILES
CHUNK = 80
NCHUNK = EPT // CHUNK
SROWS = 632
LROWS = N_NODES - (NS - 1) * SROWS

BE = 1280
NEB_REAL = N_EDGES // BE
NEB = E_PAD // BE


def _ea_body(x_ref, we_ref, be_ref, o_ref):
    i = pl.program_id(0)
    v = jnp.dot(x_ref[...], we_ref[...], preferred_element_type=jnp.float32)
    v = v + be_ref[...][None, :]
    rows = i * BE + lax.broadcasted_iota(jnp.int32, (BE, HIDDEN), 0)
    o_ref[...] = jnp.where(rows < N_EDGES, v, -1e30)


def _edge_linear(edge_attr, We, be):
    return pl.pallas_call(
        _ea_body,
        grid=(NEB,),
        in_specs=[
            pl.BlockSpec((BE, EDGE_DIM), lambda i: (jnp.minimum(i, NEB_REAL - 1), 0)),
            pl.BlockSpec((EDGE_DIM, HIDDEN), lambda i: (0, 0)),
            pl.BlockSpec((HIDDEN,), lambda i: (0,)),
        ],
        out_specs=pl.BlockSpec((BE, HIDDEN), lambda i: (i, 0)),
        out_shape=jax.ShapeDtypeStruct((E_PAD, HIDDEN), jnp.float32),
    )(edge_attr, We, be)


def _sc_body(h_hbm, init, idxb, ea, out, z_sh,
             ix0, ix1, ix2, ix3, ea0, ea1, ga0, ga1,
             s_ix0, s_ix1, s_ix2, s_ix3, s_ea0, s_ea1, s_g0, s_g1, s_sc0, s_sc1):
    c = lax.axis_index("c")
    s = lax.axis_index("s")
    wid = c * NS + s
    r0 = s * SROWS
    ix = (ix0, ix1, ix2, ix3)
    s_ix = (s_ix0, s_ix1, s_ix2, s_ix3)
    eab = (ea0, ea1)
    gab = (ga0, ga1)
    s_ea = (s_ea0, s_ea1)
    s_g = (s_g0, s_g1)
    s_sc = (s_sc0, s_sc1)

    @pl.when(s < NS - 1)
    def _stage_full():
        pltpu.sync_copy(init.at[c, pl.ds(r0, SROWS)], z_sh.at[pl.ds(r0, SROWS)])

    @pl.when(s == NS - 1)
    def _stage_last():
        pltpu.sync_copy(init.at[c, pl.ds(r0, LROWS)], z_sh.at[pl.ds(r0, LROWS)])

    plsc.subcore_barrier()

    def idx_cp(cc, j):
        return pltpu.make_async_copy(idxb.at[wid * NCHUNK + cc], ix[j], s_ix[j])

    def ea_cp(cc, b2):
        return pltpu.make_async_copy(
            ea.at[pl.ds(wid * EPT + cc * CHUNK, CHUNK)], eab[b2], s_ea[b2])

    def gat_cp(j, b2):
        return pltpu.make_async_copy(h_hbm.at[ix[j].at[0]], gab[b2], s_g[b2])

    def sc_desc(j, b2):
        return pltpu.make_async_copy(gab[b2], z_sh.at[ix[j].at[1]], s_sc[b2])

    idx_cp(0, 0).start()
    idx_cp(1, 1).start()
    idx_cp(2, 2).start()
    ea_cp(0, 0).start()
    ea_cp(1, 1).start()
    idx_cp(0, 0).wait()
    gat_cp(0, 0).start()

    def quad_body(i, carry):
        for b in range(4):
            cc = 4 * i + b
            b2 = b % 2
            ob2 = 1 - b2
            ea_cp(cc, b2).wait()
            gat_cp(b, b2).wait()

            def vbody(r, acc):
                g_r = gab[b2].at[r]
                e_r = eab[b2].at[r]
                for k in range(HIDDEN // 16):
                    sl = pl.ds(k * 16, 16)
                    g_r[sl] = jnp.maximum(g_r[sl] + e_r[sl], 0.0)
                return acc
            lax.fori_loop(0, CHUNK, vbody, 0, unroll=2)

            pltpu.async_copy(gab[b2], z_sh.at[ix[b].at[1]], s_sc[b2], add=True)

            @pl.when(cc + 2 < NCHUNK)
            def _fire_ea():
                ea_cp(cc + 2, b2).start()

            if b == 0:
                @pl.when(cc >= 1)
                def _wait_sc0():
                    sc_desc(3, ob2).wait()
            else:
                sc_desc(b - 1, ob2).wait()

            @pl.when(cc + 3 < NCHUNK)
            def _fire_idx():
                idx_cp(cc + 3, (b + 3) % 4).start()

            @pl.when(cc + 1 < NCHUNK)
            def _fire_gat():
                idx_cp(cc + 1, (b + 1) % 4).wait()
                gat_cp((b + 1) % 4, ob2).start()
        return carry

    lax.fori_loop(0, NCHUNK // 4, quad_body, 0)
    sc_desc(3, 1).wait()
    plsc.subcore_barrier()

    @pl.when(s < NS - 1)
    def _out_full():
        pltpu.sync_copy(z_sh.at[pl.ds(r0, SROWS)], out.at[c, pl.ds(r0, SROWS)])

    @pl.when(s == NS - 1)
    def _out_last():
        pltpu.sync_copy(z_sh.at[pl.ds(r0, LROWS)], out.at[c, pl.ds(r0, LROWS)])


def _sc_aggregate(h, init, idxb, ea):
    mesh = plsc.VectorSubcoreMesh(core_axis_name="c", subcore_axis_name="s")
    return pl.kernel(
        _sc_body,
        out_type=jax.ShapeDtypeStruct((NC, N_NODES, HIDDEN), jnp.float32),
        mesh=mesh,
        scratch_types=[
            pltpu.VMEM_SHARED((N_NODES, HIDDEN), jnp.float32),
            pltpu.VMEM((2, CHUNK), jnp.int32),
            pltpu.VMEM((2, CHUNK), jnp.int32),
            pltpu.VMEM((2, CHUNK), jnp.int32),
            pltpu.VMEM((2, CHUNK), jnp.int32),
            pltpu.VMEM((CHUNK, HIDDEN), jnp.float32),
            pltpu.VMEM((CHUNK, HIDDEN), jnp.float32),
            pltpu.VMEM((CHUNK, HIDDEN), jnp.float32),
            pltpu.VMEM((CHUNK, HIDDEN), jnp.float32),
        ] + [pltpu.SemaphoreType.DMA] * 10,
    )(h, init, idxb, ea)


def _mlp_body(pa_ref, pb_ref, h_ref, w1_ref, g1_ref, b1_ref, w2_ref,
              g2_ref, b2_ref, o_ref):
    z = pa_ref[0] + pb_ref[0]
    y = jnp.dot(z, w1_ref[...], preferred_element_type=jnp.float32)
    mu = jnp.mean(y, axis=0, keepdims=True)
    d = y - mu
    var = jnp.mean(d * d, axis=0, keepdims=True)
    y = d * lax.rsqrt(var + BN_EPS) * g1_ref[...][None, :] + b1_ref[...][None, :]
    y = jnp.maximum(y, 0.0)
    y = jnp.dot(y, w2_ref[...], preferred_element_type=jnp.float32)
    mu = jnp.mean(y, axis=0, keepdims=True)
    d = y - mu
    var = jnp.mean(d * d, axis=0, keepdims=True)
    y = d * lax.rsqrt(var + BN_EPS) * g2_ref[...][None, :] + b2_ref[...][None, :]
    o_ref[...] = jnp.maximum(y, 0.0) + h_ref[...]


def _mlp(z2, h, W1, g1, b1, W2, g2, b2):
    full = lambda *shape: pl.BlockSpec(shape, lambda: (0,) * len(shape))
    return pl.pallas_call(
        _mlp_body,
        in_specs=[
            full(1, N_NODES, HIDDEN), full(1, N_NODES, HIDDEN),
            full(N_NODES, HIDDEN),
            full(HIDDEN, 2 * HIDDEN), full(2 * HIDDEN), full(2 * HIDDEN),
            full(2 * HIDDEN, HIDDEN), full(HIDDEN), full(HIDDEN),
        ],
        out_specs=full(N_NODES, HIDDEN),
        out_shape=jax.ShapeDtypeStruct((N_NODES, HIDDEN), jnp.float32),
    )(z2[0:1], z2[1:2], h, W1, g1, b1, W2, g2, b2)


def kernel(h, edge_index, edge_attr, We, be, eps, W1, g1, b1, W2, g2, b2):
    src = edge_index[0].astype(jnp.int32)
    dst = edge_index[1].astype(jnp.int32)
    pad = E_PAD - N_EDGES
    zpad = jnp.zeros((pad,), jnp.int32)
    srcr = jnp.concatenate([src, zpad]).reshape(TILES * NCHUNK, 1, CHUNK)
    dstr = jnp.concatenate([dst, zpad]).reshape(TILES * NCHUNK, 1, CHUNK)
    idxb = jnp.concatenate([srcr, dstr], axis=1)

    init = jnp.stack([(1.0 + eps) * h, jnp.zeros_like(h)])

    ea = _edge_linear(edge_attr, We, be)
    z2 = _sc_aggregate(h, init, idxb, ea)
    return _mlp(z2, h, W1, g1, b1, W2, g2, b2)
